# baseline (speedup 1.0000x reference)
import functools

import jax
import jax.numpy as jnp
from jax import lax
from jax.experimental import pallas as pl
from jax.experimental.pallas import tpu as pltpu
from jax.experimental.pallas import tpu_sc as plsc

NC = 2
NS = 16
L = 16
NW = NC * NS

EMBED_DIM = 32
BATCH = 16384
BW = BATCH // NW
CHUNK = 128
NCH = BW // CHUNK
GPC = CHUNK // L


def _nsqrt(x):
    xi = lax.bitcast_convert_type(x, jnp.int32)
    yi = lax.shift_right_logical(xi, 1) + jnp.int32(0x1FBD1DF5)
    y = lax.bitcast_convert_type(yi, jnp.float32)
    for _ in range(3):
        y = 0.5 * (y + x / y)
    return -y


def _body(h_hbm, t_hbm, r_hbm, ent_hbm, rel_hbm, out_hbm,
          h_idx, t_idx, r_idx, rows_h, rows_t, rows_r, out_v, sem):
    wid = lax.axis_index("s") * NC + lax.axis_index("c")
    base = wid * BW

    for c in range(NCH):
        off = base + c * CHUNK
        pltpu.sync_copy(h_hbm.at[pl.ds(off, CHUNK)], h_idx.at[c])
        pltpu.sync_copy(t_hbm.at[pl.ds(off, CHUNK)], t_idx.at[c])
        pltpu.sync_copy(r_hbm.at[pl.ds(off, CHUNK)], r_idx.at[c])

    copies = []
    for c in range(NCH):
        copies.append(pltpu.async_copy(ent_hbm.at[h_idx.at[c]], rows_h.at[c], sem))
        copies.append(pltpu.async_copy(ent_hbm.at[t_idx.at[c]], rows_t.at[c], sem))
        copies.append(pltpu.async_copy(rel_hbm.at[r_idx.at[c]], rows_r.at[c], sem))
    for cp in copies:
        cp.wait()

    lane = lax.iota(jnp.int32, L)

    for c in range(NCH):
        def group_body(g, carry):
            rb = g * L
            acc = jnp.zeros((L,), jnp.float32)
            for j in range(L):
                i = rb + j
                h0 = rows_h[c, i, 0:L]
                h1 = rows_h[c, i, L:EMBED_DIM]
                t0 = rows_t[c, i, 0:L]
                t1 = rows_t[c, i, L:EMBED_DIM]
                r0 = rows_r[c, i, 0:L]
                r1 = rows_r[c, i, L:EMBED_DIM]
                d0 = (h0 - t0) + r0
                d1 = (h1 - t1) + r1
                sq = d0 * d0 + d1 * d1
                acc = jnp.where(lane == j, jnp.sum(sq), acc)
            out_v[pl.ds(c * CHUNK + rb, L)] = _nsqrt(acc + 1e-12)
            return carry

        lax.fori_loop(0, GPC, group_body, 0)

    pltpu.sync_copy(out_v, out_hbm.at[pl.ds(base, BW)])


_sc_call = functools.partial(
    pl.kernel,
    mesh=plsc.VectorSubcoreMesh(core_axis_name="c", subcore_axis_name="s"),
    out_type=jax.ShapeDtypeStruct((BATCH,), jnp.float32),
    compiler_params=pltpu.CompilerParams(
        needs_layout_passes=False, use_tc_tiling_on_sc=False
    ),
    scratch_types=[
        pltpu.VMEM((NCH, CHUNK), jnp.int32),
        pltpu.VMEM((NCH, CHUNK), jnp.int32),
        pltpu.VMEM((NCH, CHUNK), jnp.int32),
        pltpu.VMEM((NCH, CHUNK, EMBED_DIM), jnp.float32),
        pltpu.VMEM((NCH, CHUNK, EMBED_DIM), jnp.float32),
        pltpu.VMEM((NCH, CHUNK, EMBED_DIM), jnp.float32),
        pltpu.VMEM((BW,), jnp.float32),
        pltpu.SemaphoreType.DMA,
    ],
)(_body)


@jax.jit
def kernel(triples, entity_table, relation_table):
    h = triples[0]
    t = triples[1]
    r = triples[2]
    return _sc_call(h, t, r, entity_table, relation_table)
